# baseline (speedup 1.0000x reference)
import functools

import jax
import jax.numpy as jnp
from jax import lax
from jax.experimental import pallas as pl
from jax.experimental.pallas import tpu as pltpu
from jax.experimental.pallas import tpu_sc as plsc

NC, NS, L = 2, 16, 16
NW = NC * NS
CH = 128


def _sc_deg(dst_p, npad, ept):
    nchunk16 = ept // L
    mesh = plsc.VectorSubcoreMesh(core_axis_name="c", subcore_axis_name="s",
                                  num_cores=NC, num_subcores=NS)

    @functools.partial(
        pl.kernel,
        out_type=jax.ShapeDtypeStruct((NW * npad,), jnp.float32),
        mesh=mesh,
        compiler_params=pltpu.CompilerParams(needs_layout_passes=False),
        scratch_types=[
            pltpu.VMEM((ept,), jnp.int32),
            pltpu.VMEM((npad,), jnp.float32),
        ],
    )
    def deg_kernel(dst_hbm, out_hbm, idx_v, deg_v):
        cid = lax.axis_index("c")
        sid = lax.axis_index("s")
        wid = sid * NC + cid
        pltpu.sync_copy(dst_hbm.at[pl.ds(wid * ept, ept)], idx_v)
        zeros = jnp.zeros((L,), jnp.float32)
        ones = jnp.ones((L,), jnp.float32)

        def zbody(i, carry):
            deg_v[pl.ds(i * L, L)] = zeros
            return carry
        lax.fori_loop(0, npad // L, zbody, 0)

        def body(i, carry):
            idx = idx_v[pl.ds(i * L, L)]
            plsc.addupdate_scatter(deg_v, [idx], ones)
            return carry
        lax.fori_loop(0, nchunk16, body, 0)
        pltpu.sync_copy(deg_v, out_hbm.at[pl.ds(wid * npad, npad)])

    return deg_kernel(dst_p)


def _tc_dinv(deg_parts):
    npad = deg_parts.shape[1]

    def body(deg_ref, out_ref):
        deg = jnp.sum(deg_ref[...], axis=0) + 1.0
        out_ref[...] = lax.rsqrt(deg)

    return pl.pallas_call(
        body,
        out_shape=jax.ShapeDtypeStruct((npad,), jnp.float32),
    )(deg_parts)


def _tc_xs(x, dinv_col):
    def body(x_ref, d_ref, out_ref):
        out_ref[...] = x_ref[...] * d_ref[...]

    return pl.pallas_call(
        body,
        out_shape=jax.ShapeDtypeStruct(x.shape, jnp.float32),
    )(x, dinv_col)


def _sc_agg(src2d, dst2d, xs, nrows_sh):
    n, d = xs.shape
    ncht = src2d.shape[0] // NS
    NPH = 2
    nch = ncht // NPH
    rows_per_tile_out = nrows_sh // NS
    zch = nrows_sh // (NS * CH)
    mesh = plsc.VectorSubcoreMesh(core_axis_name="c", subcore_axis_name="s",
                                  num_cores=NC, num_subcores=NS)

    G = 8
    NB = nch // G

    @functools.partial(
        pl.kernel,
        out_type=jax.ShapeDtypeStruct((NC, nrows_sh, d), jnp.float32),
        mesh=mesh,
        compiler_params=pltpu.CompilerParams(needs_layout_passes=False),
        scratch_types=[
            pltpu.VMEM((G, CH), jnp.int32),
            pltpu.VMEM((G, CH), jnp.int32),
            pltpu.VMEM((nch, CH), jnp.int32),
            pltpu.VMEM((CH, d), jnp.float32),
            pltpu.VMEM((CH, d), jnp.float32),
            pltpu.VMEM_SHARED((nrows_sh, d), jnp.float32),
            pltpu.SemaphoreType.DMA((2,)),
            pltpu.SemaphoreType.DMA((2,)),
        ],
    )
    def agg_kernel(src_hbm, dst_hbm, xs_hbm, out_hbm,
                   sxA, sxB, didx_v, r0, r1, agg_sh, lsem, gsem):
        slots = (sxA, sxB)
        rows = (r0, r1)
        cid = lax.axis_index("c")
        sid = lax.axis_index("s")

        def sidx_load_desc(ph, m, sl):
            return pltpu.make_async_copy(
                src_hbm.at[pl.ds((sid * NPH + ph) * nch + m * G, G)],
                slots[sl], lsem.at[sl])

        def gather_desc(slot, row, b):
            return pltpu.make_async_copy(xs_hbm.at[slot.at[row]], rows[b],
                                         gsem.at[b])

        zeros = jnp.zeros((L,), jnp.float32)

        def zv(k, carry):
            r0[k >> 3, pl.ds((k & 7) * L, L)] = zeros
            return carry
        lax.fori_loop(0, CH * (d // L), zv, 0)

        def zs(k, carry):
            pltpu.sync_copy(r0, agg_sh.at[pl.ds((sid * zch + k) * CH, CH)])
            return carry
        lax.fori_loop(0, zch, zs, 0)
        plsc.subcore_barrier()

        def run_phase(ph):
            pltpu.sync_copy(
                dst_hbm.at[pl.ds((sid * NPH + ph) * nch, nch)], didx_v)
            sidx_load_desc(ph, 0, 0).start()
            sidx_load_desc(ph, 1, 1).start()
            sidx_load_desc(ph, 0, 0).wait()
            gather_desc(sxA, 0, 0).start()
            gather_desc(sxA, 1, 1).start()

            def pair(p, carry):
                for mb in range(2):
                    m = 2 * p + mb
                    cur, nxt = slots[mb], slots[1 - mb]
                    for j in range(G):
                        i = m * G + j
                        b = j & 1
                        gather_desc(cur, j, b).wait()
                        pltpu.sync_copy(rows[b], agg_sh.at[didx_v.at[i]],
                                        add=True)
                        if j < G - 2:
                            gather_desc(cur, j + 2, b).start()
                        else:
                            @pl.when(m + 1 < NB)
                            def _():
                                if j == G - 2:
                                    sidx_load_desc(ph, m + 1, 1 - mb).wait()
                                gather_desc(nxt, j + 2 - G, b).start()
                    @pl.when(m + 2 < NB)
                    def _():
                        sidx_load_desc(ph, m + 2, mb).start()
                return carry
            lax.fori_loop(0, NB // 2, pair, 0)

        @pl.when(cid == 0)
        def _():
            for ph in range(NPH):
                run_phase(ph)
        plsc.subcore_barrier()

        pltpu.sync_copy(
            agg_sh.at[pl.ds(sid * rows_per_tile_out, rows_per_tile_out)],
            out_hbm.at[cid, pl.ds(sid * rows_per_tile_out, rows_per_tile_out)])

    return agg_kernel(src2d, dst2d, xs)


def _tc_final(agg, xs, dinv_col, W, b2, al2):
    n, d = xs.shape
    d_out = W.shape[1]
    blk = 1000

    def body(a_ref, xs_ref, d_ref, w_ref, b_ref, al_ref, out_ref):
        t = (a_ref[0] + a_ref[1] + xs_ref[...]) * d_ref[...]
        o = jnp.dot(t, w_ref[...], preferred_element_type=jnp.float32)
        o = o + b_ref[...]
        out_ref[...] = jnp.where(o > 0, o, al_ref[...] * o)

    return pl.pallas_call(
        body,
        grid=(n // blk,),
        in_specs=[
            pl.BlockSpec((NC, blk, d), lambda i: (0, i, 0)),
            pl.BlockSpec((blk, d), lambda i: (i, 0)),
            pl.BlockSpec((blk, 1), lambda i: (i, 0)),
            pl.BlockSpec((d, d_out), lambda i: (0, 0)),
            pl.BlockSpec((1, d_out), lambda i: (0, 0)),
            pl.BlockSpec((1, d_out), lambda i: (0, 0)),
        ],
        out_specs=pl.BlockSpec((blk, d_out), lambda i: (i, 0)),
        out_shape=jax.ShapeDtypeStruct((n, d_out), jnp.float32),
    )(agg, xs, dinv_col, W, b2, al2)


def kernel(x, edge_index, W, b, alpha):
    n, d_in = x.shape
    e = edge_index.shape[1]

    ept = -(-e // (NW * 8 * CH)) * 8 * CH
    ep = ept * NW
    nch = ept // CH
    npad = -(-(n + 1) // CH) * CH
    nrows_sh = 10240

    src = edge_index[0]
    dst = edge_index[1]
    pad = ep - e
    src_p = jnp.concatenate([src, jnp.zeros((pad,), jnp.int32)])
    dst_p = jnp.concatenate([dst, jnp.full((pad,), n, jnp.int32)])
    src2d = src_p.reshape(NW * nch, CH)
    dst2d = dst_p.reshape(NW * nch, CH)

    deg_parts = _sc_deg(dst_p, npad, ept).reshape(NW, npad)
    dinv_flat = _tc_dinv(deg_parts)
    dinv_col = dinv_flat[:n, None]
    xs = _tc_xs(x, dinv_col)
    agg = _sc_agg(src2d, dst2d, xs, nrows_sh)
    out = _tc_final(agg, xs, dinv_col, W,
                    b.reshape(1, -1), alpha.reshape(1, -1))
    return out


# speedup vs baseline: 1.1978x; 1.1978x over previous
import functools

import jax
import jax.numpy as jnp
from jax import lax
from jax.experimental import pallas as pl
from jax.experimental.pallas import tpu as pltpu
from jax.experimental.pallas import tpu_sc as plsc

NC, NS, L = 2, 16, 16
NW = NC * NS
CH = 128


def _sc_deg(dst_p, npad, ept):
    nchunk16 = ept // L
    mesh = plsc.VectorSubcoreMesh(core_axis_name="c", subcore_axis_name="s",
                                  num_cores=NC, num_subcores=NS)

    @functools.partial(
        pl.kernel,
        out_type=jax.ShapeDtypeStruct((NW * npad,), jnp.float32),
        mesh=mesh,
        compiler_params=pltpu.CompilerParams(needs_layout_passes=False),
        scratch_types=[
            pltpu.VMEM((ept,), jnp.int32),
            pltpu.VMEM((npad,), jnp.float32),
        ],
    )
    def deg_kernel(dst_hbm, out_hbm, idx_v, deg_v):
        cid = lax.axis_index("c")
        sid = lax.axis_index("s")
        wid = sid * NC + cid
        pltpu.sync_copy(dst_hbm.at[pl.ds(wid * ept, ept)], idx_v)
        zeros = jnp.zeros((L,), jnp.float32)
        ones = jnp.ones((L,), jnp.float32)

        def zbody(i, carry):
            deg_v[pl.ds(i * L, L)] = zeros
            return carry
        lax.fori_loop(0, npad // L, zbody, 0)

        def body(i, carry):
            idx = idx_v[pl.ds(i * L, L)]
            plsc.addupdate_scatter(deg_v, [idx], ones)
            return carry
        lax.fori_loop(0, nchunk16, body, 0)
        pltpu.sync_copy(deg_v, out_hbm.at[pl.ds(wid * npad, npad)])

    return deg_kernel(dst_p)


def _tc_dinv(deg_parts):
    npad = deg_parts.shape[1]

    def body(deg_ref, out_ref):
        deg = jnp.sum(deg_ref[...], axis=0) + 1.0
        out_ref[...] = lax.rsqrt(deg)

    return pl.pallas_call(
        body,
        out_shape=jax.ShapeDtypeStruct((npad,), jnp.float32),
    )(deg_parts)


def _tc_xs(x, dinv_col):
    def body(x_ref, d_ref, out_ref):
        out_ref[...] = x_ref[...] * d_ref[...]

    return pl.pallas_call(
        body,
        out_shape=jax.ShapeDtypeStruct(x.shape, jnp.float32),
    )(x, dinv_col)


def _sc_agg(src2d, dst2d, xs, nrows_sh):
    n, d = xs.shape
    ncht = src2d.shape[0] // NS
    rows_per_tile_out = nrows_sh // NS
    zch = nrows_sh // (NS * CH)
    mesh = plsc.VectorSubcoreMesh(core_axis_name="c", subcore_axis_name="s",
                                  num_cores=NC, num_subcores=NS)

    NCH0 = (ncht * 3) // 4
    NCH0 = (NCH0 // 8) * 8
    NCH1 = ncht - NCH0
    G = 4
    nch_max = max(NCH0, NCH1)

    @functools.partial(
        pl.kernel,
        out_type=jax.ShapeDtypeStruct((NC, nrows_sh, d), jnp.float32),
        mesh=mesh,
        compiler_params=pltpu.CompilerParams(needs_layout_passes=False),
        scratch_types=[
            pltpu.VMEM((G, CH), jnp.int32),
            pltpu.VMEM((G, CH), jnp.int32),
            pltpu.VMEM((nch_max, CH), jnp.int32),
            pltpu.VMEM((CH, d), jnp.float32),
            pltpu.VMEM((CH, d), jnp.float32),
            pltpu.VMEM_SHARED((nrows_sh, d), jnp.float32),
            pltpu.SemaphoreType.DMA((2,)),
            pltpu.SemaphoreType.DMA((2,)),
        ],
    )
    def agg_kernel(src_hbm, dst_hbm, xs_hbm, out_hbm,
                   sxA, sxB, didx_v, r0, r1, agg_sh, lsem, gsem):
        slots = (sxA, sxB)
        rows = (r0, r1)
        cid = lax.axis_index("c")
        sid = lax.axis_index("s")
        base = jnp.where(cid == 0, sid * NCH0, NS * NCH0 + sid * NCH1)

        def sidx_load_desc(m, sl):
            return pltpu.make_async_copy(
                src_hbm.at[pl.ds(base + m * G, G)], slots[sl],
                lsem.at[sl])

        def gather_desc(slot, row, b):
            return pltpu.make_async_copy(xs_hbm.at[slot.at[row]], rows[b],
                                         gsem.at[b])

        sidx_load_desc(0, 0).start()

        zeros = jnp.zeros((L,), jnp.float32)

        def zv(k, carry):
            r0[k >> 3, pl.ds((k & 7) * L, L)] = zeros
            return carry
        lax.fori_loop(0, CH * (d // L), zv, 0)

        def zs(k, carry):
            pltpu.sync_copy(r0, agg_sh.at[pl.ds(sid * rows_per_tile_out
                                                + k * CH, CH)])
            return carry
        lax.fori_loop(0, zch, zs, 0)

        ztail = rows_per_tile_out - zch * CH
        if ztail:
            pltpu.sync_copy(
                r0.at[pl.ds(0, ztail)],
                agg_sh.at[pl.ds(sid * rows_per_tile_out + zch * CH, ztail)])
        plsc.subcore_barrier()

        def run_core(nch):
            NB = nch // G
            pltpu.sync_copy(dst_hbm.at[pl.ds(base, nch)],
                            didx_v.at[pl.ds(0, nch)])
            sidx_load_desc(1, 1).start()
            sidx_load_desc(0, 0).wait()
            gather_desc(sxA, 0, 0).start()
            gather_desc(sxA, 1, 1).start()

            def pair(p, carry):
                for mb in range(2):
                    m = 2 * p + mb
                    cur, nxt = slots[mb], slots[1 - mb]
                    for j in range(G):
                        i = m * G + j
                        b = j & 1
                        gather_desc(cur, j, b).wait()
                        pltpu.sync_copy(rows[b], agg_sh.at[didx_v.at[i]],
                                        add=True)
                        if j < G - 2:
                            gather_desc(cur, j + 2, b).start()
                        else:
                            @pl.when(m + 1 < NB)
                            def _():
                                if j == G - 2:
                                    sidx_load_desc(m + 1, 1 - mb).wait()
                                gather_desc(nxt, j + 2 - G, b).start()
                    @pl.when(m + 2 < NB)
                    def _():
                        sidx_load_desc(m + 2, mb).start()
                return carry
            lax.fori_loop(0, NB // 2, pair, 0)

        @pl.when(cid == 0)
        def _():
            run_core(NCH0)

        @pl.when(cid == 1)
        def _():
            run_core(NCH1)
        plsc.subcore_barrier()

        pltpu.sync_copy(
            agg_sh.at[pl.ds(sid * rows_per_tile_out, rows_per_tile_out)],
            out_hbm.at[cid, pl.ds(sid * rows_per_tile_out, rows_per_tile_out)])

    return agg_kernel(src2d, dst2d, xs)


def _tc_final(agg, xs, dinv_col, W, b2, al2):
    n, d = xs.shape
    d_out = W.shape[1]
    blk = 1000

    def body(a_ref, xs_ref, d_ref, w_ref, b_ref, al_ref, out_ref):
        t = (a_ref[0] + a_ref[1] + xs_ref[...]) * d_ref[...]
        o = jnp.dot(t, w_ref[...], preferred_element_type=jnp.float32)
        o = o + b_ref[...]
        out_ref[...] = jnp.where(o > 0, o, al_ref[...] * o)

    return pl.pallas_call(
        body,
        grid=(n // blk,),
        in_specs=[
            pl.BlockSpec((NC, blk, d), lambda i: (0, i, 0)),
            pl.BlockSpec((blk, d), lambda i: (i, 0)),
            pl.BlockSpec((blk, 1), lambda i: (i, 0)),
            pl.BlockSpec((d, d_out), lambda i: (0, 0)),
            pl.BlockSpec((1, d_out), lambda i: (0, 0)),
            pl.BlockSpec((1, d_out), lambda i: (0, 0)),
        ],
        out_specs=pl.BlockSpec((blk, d_out), lambda i: (i, 0)),
        out_shape=jax.ShapeDtypeStruct((n, d_out), jnp.float32),
    )(agg, xs, dinv_col, W, b2, al2)


def kernel(x, edge_index, W, b, alpha):
    n, d_in = x.shape
    e = edge_index.shape[1]

    ept = -(-e // (NW * 8 * CH)) * 8 * CH
    ep = ept * NW
    nch = ept // CH
    npad = -(-(n + 1) // CH) * CH
    nrows_sh = 10112

    src = edge_index[0]
    dst = edge_index[1]
    pad = ep - e
    src_p = jnp.concatenate([src, jnp.zeros((pad,), jnp.int32)])
    dst_p = jnp.concatenate([dst, jnp.full((pad,), n, jnp.int32)])
    src2d = src_p.reshape(NW * nch, CH)
    dst2d = dst_p.reshape(NW * nch, CH)

    deg_parts = _sc_deg(dst_p, npad, ept).reshape(NW, npad)
    dinv_flat = _tc_dinv(deg_parts)
    dinv_col = dinv_flat[:n, None]
    xs = _tc_xs(x, dinv_col)
    agg = _sc_agg(src2d, dst2d, xs, nrows_sh)
    out = _tc_final(agg, xs, dinv_col, W,
                    b.reshape(1, -1), alpha.reshape(1, -1))
    return out
